# Initial kernel scaffold; baseline (speedup 1.0000x reference)
import jax
import jax.numpy as jnp
from jax.experimental import pallas as pl


def kernel(classifications, regressions, anchors, annotations):
    raise NotImplementedError("write your pallas kernel here")


# speedup vs baseline: 1.5617x; 1.5617x over previous
import functools

import jax
import jax.numpy as jnp
from jax.experimental import pallas as pl
from jax.experimental.pallas import tpu as pltpu

_NV = 3
_MAX_POS = 0.05
_MAX_ANG = 0.1
_ALPHA = 0.95


def _body(cls_ref, reg_ref, anch_ref, ann_ref, annT_ref, out_ref):
    k = pl.program_id(1)
    p = jnp.clip(cls_ref[0], 1e-4, 1.0 - 1e-4)
    reg = reg_ref[0]
    anch = anch_ref[0]
    ann = ann_ref[0]
    annT = annT_ref[0]

    tx = annT[0:1, :]
    ty = annT[1:2, :]
    ax = anch[:, 0:1]
    ay = anch[:, 1:2]
    dx = ax - tx
    dy = ay - ty
    d2 = dx * dx + dy * dy
    m = jnp.min(d2, axis=1)
    iota = jax.lax.broadcasted_iota(jnp.int32, d2.shape, 1)
    big = jnp.iinfo(jnp.int32).max
    idx = jnp.min(jnp.where(d2 == m[:, None], iota, big), axis=1)
    onehot = (iota == idx[:, None]).astype(jnp.float32)
    assigned = jax.lax.dot_general(
        onehot, ann, (((1,), (0,)), ((), ())),
        preferred_element_type=jnp.float32)

    a_dist = jnp.abs(anch[:, 2] - assigned[:, 2])
    pos = (m < _MAX_POS**2) & (a_dist < _MAX_ANG)
    neg = (m >= (1.5 * _MAX_POS) ** 2) | (a_dist >= 1.5 * _MAX_ANG)
    w = (pos | neg).astype(jnp.float32)
    posf = pos.astype(jnp.float32)

    tcls = assigned[:, 3].astype(jnp.int32)
    ciota = jax.lax.broadcasted_iota(jnp.int32, p.shape, 1)
    is_t1 = (ciota == tcls[:, None]) & pos[:, None]
    l0 = (1.0 - _ALPHA) * p * p * (-jnp.log1p(-p))
    l1 = _ALPHA * (1.0 - p) * (1.0 - p) * (-jnp.log(p))
    cls_sum = jnp.sum(jnp.where(is_t1, l1, l0) * w[:, None])

    d = reg - (assigned[:, :_NV] - anch)
    ad = jnp.abs(d)
    sl1 = jnp.where(ad <= 1.0 / 9.0, 4.5 * ad * ad, ad - 0.5 / 9.0)
    xy = jnp.sum((sl1[:, 0] + sl1[:, 1]) * posf)
    ang = jnp.sum(sl1[:, 2] * posf)
    npos = jnp.sum(posf)

    part = jnp.stack([cls_sum, xy, ang, npos])

    @pl.when(k == 0)
    def _():
        out_ref[0, 0, 0] = part[0]
        out_ref[0, 0, 1] = part[1]
        out_ref[0, 0, 2] = part[2]
        out_ref[0, 0, 3] = part[3]

    @pl.when(k > 0)
    def _():
        out_ref[0, 0, 0] += part[0]
        out_ref[0, 0, 1] += part[1]
        out_ref[0, 0, 2] += part[2]
        out_ref[0, 0, 3] += part[3]


@jax.jit
def kernel(classifications, regressions, anchors, annotations):
    B, N, C = classifications.shape
    M = annotations.shape[1]
    bn = 5000 if N % 5000 == 0 else N
    nb = N // bn
    annT = annotations.transpose(0, 2, 1)

    out = pl.pallas_call(
        _body,
        grid=(B, nb),
        in_specs=[
            pl.BlockSpec((1, bn, C), lambda b, k: (b, k, 0)),
            pl.BlockSpec((1, bn, _NV), lambda b, k: (b, k, 0)),
            pl.BlockSpec((1, bn, _NV), lambda b, k: (0, k, 0)),
            pl.BlockSpec((1, M, 4), lambda b, k: (b, 0, 0)),
            pl.BlockSpec((1, 4, M), lambda b, k: (b, 0, 0)),
        ],
        out_specs=pl.BlockSpec(
            (1, 1, 4), lambda b, k: (b, 0, 0), memory_space=pltpu.SMEM),
        out_shape=jax.ShapeDtypeStruct((B, 1, 4), jnp.float32),
    )(classifications, regressions, anchors, annotations, annT)

    out = out[:, 0, :]
    denom = jnp.maximum(out[:, 3], 1.0)
    return jnp.stack([
        jnp.mean(out[:, 0] / denom),
        jnp.mean(out[:, 1] / denom),
        jnp.mean(out[:, 2] / denom),
    ])


# speedup vs baseline: 16.4051x; 10.5045x over previous
import jax
import jax.numpy as jnp
from jax.experimental import pallas as pl
from jax.experimental.pallas import tpu as pltpu

_MAX_POS = 0.05
_MAX_ANG = 0.1
_ALPHA = 0.95
_POS2 = _MAX_POS * _MAX_POS
_NEG2 = (1.5 * _MAX_POS) ** 2
_ANG1 = _MAX_ANG
_ANG15 = 1.5 * _MAX_ANG
_CH = 16


def _make_body(N, C, M, RT):
    nch = RT // _CH

    def _body(cls_ref, reg_ref, anch_ref, ann_ref, out_ref):
        def chunk(i, accs):
            acc_cls, acc_xy, acc_ang, acc_np = accs
            sl = pl.ds(i * _CH, _CH)
            ax = anch_ref[0, sl, :]
            ay = anch_ref[1, sl, :]
            aa = anch_ref[2, sl, :]

            m = jnp.full((_CH, 128), 1e30, jnp.float32)
            tx = jnp.zeros((_CH, 128), jnp.float32)
            ty = jnp.zeros((_CH, 128), jnp.float32)
            ta = jnp.zeros((_CH, 128), jnp.float32)
            tc = jnp.zeros((_CH, 128), jnp.float32)
            for j in range(M):
                txj = ann_ref[0, 0, j]
                tyj = ann_ref[0, 1, j]
                taj = ann_ref[0, 2, j]
                tcj = ann_ref[0, 3, j]
                bj = jnp.where(tcj != -1.0, 0.0, 1e9)
                dx = ax - txj
                dy = ay - tyj
                d2 = dx * dx + (dy * dy + bj)
                pred = d2 < m
                m = jnp.where(pred, d2, m)
                tx = jnp.where(pred, txj, tx)
                ty = jnp.where(pred, tyj, ty)
                ta = jnp.where(pred, taj, ta)
                tc = jnp.where(pred, tcj, tc)

            row = jax.lax.broadcasted_iota(jnp.int32, (_CH, 128), 0)
            lane = jax.lax.broadcasted_iota(jnp.int32, (_CH, 128), 1)
            gmask = (i * (_CH * 128) + row * 128 + lane) < N

            a_dist = jnp.abs(aa - ta)
            posm = (m < _POS2) & (a_dist < _ANG1) & gmask
            w = (((m >= _NEG2) | (a_dist >= _ANG15)) | posm) & gmask
            icls = tc.astype(jnp.int32)

            zero = jnp.zeros((_CH, 128), jnp.float32)
            s0 = zero
            corr = zero
            for c in range(C):
                p = jnp.clip(cls_ref[c, 0, sl, :], 1e-4, 1.0 - 1e-4)
                l0 = (p * p) * jnp.log1p(-p) * (-(1.0 - _ALPHA))
                omp = 1.0 - p
                l1 = (omp * omp) * jnp.log(p) * (-_ALPHA)
                s0 = s0 + l0
                corr = corr + jnp.where(posm & (icls == c), l1 - l0, zero)
            acc_cls = acc_cls + jnp.where(w, s0, zero) + corr

            sl1 = []
            for c, (a_c, t_c) in enumerate(((ax, tx), (ay, ty), (aa, ta))):
                d = reg_ref[c, 0, sl, :] - (t_c - a_c)
                ad = jnp.abs(d)
                sl1.append(jnp.where(ad <= 1.0 / 9.0, 4.5 * ad * ad,
                                     ad - 0.5 / 9.0))
            acc_xy = acc_xy + jnp.where(posm, sl1[0] + sl1[1], zero)
            acc_ang = acc_ang + jnp.where(posm, sl1[2], zero)
            acc_np = acc_np + jnp.where(posm, 1.0, 0.0)
            return (acc_cls, acc_xy, acc_ang, acc_np)

        z = jnp.zeros((_CH, 128), jnp.float32)
        acc_cls, acc_xy, acc_ang, acc_np = jax.lax.fori_loop(
            0, nch, chunk, (z, z, z, z))
        out_ref[0, 0, 0] = jnp.sum(acc_cls)
        out_ref[0, 0, 1] = jnp.sum(acc_xy)
        out_ref[0, 0, 2] = jnp.sum(acc_ang)
        out_ref[0, 0, 3] = jnp.sum(acc_np)

    return _body


@jax.jit
def kernel(classifications, regressions, anchors, annotations):
    B, N, C = classifications.shape
    M = annotations.shape[1]
    RT = -(-N // 128)
    if RT % _CH:
        RT += _CH - RT % _CH
    NP = RT * 128

    clsT = jnp.pad(classifications.transpose(2, 0, 1),
                   ((0, 0), (0, 0), (0, NP - N)),
                   constant_values=0.5).reshape(C, B, RT, 128)
    regT = jnp.pad(regressions.transpose(2, 0, 1),
                   ((0, 0), (0, 0), (0, NP - N))).reshape(3, B, RT, 128)
    anchT = jnp.pad(anchors[0].transpose(1, 0),
                    ((0, 0), (0, NP - N)),
                    constant_values=9.0).reshape(3, RT, 128)
    annT = annotations.transpose(0, 2, 1)

    out = pl.pallas_call(
        _make_body(N, C, M, RT),
        grid=(B,),
        in_specs=[
            pl.BlockSpec((C, 1, RT, 128), lambda b: (0, b, 0, 0)),
            pl.BlockSpec((3, 1, RT, 128), lambda b: (0, b, 0, 0)),
            pl.BlockSpec((3, RT, 128), lambda b: (0, 0, 0)),
            pl.BlockSpec((1, 4, M), lambda b: (b, 0, 0),
                         memory_space=pltpu.SMEM),
        ],
        out_specs=pl.BlockSpec(
            (1, 1, 4), lambda b: (b, 0, 0), memory_space=pltpu.SMEM),
        out_shape=jax.ShapeDtypeStruct((B, 1, 4), jnp.float32),
    )(clsT, regT, anchT, annT)

    out = out[:, 0, :]
    denom = jnp.maximum(out[:, 3], 1.0)
    return jnp.stack([
        jnp.mean(out[:, 0] / denom),
        jnp.mean(out[:, 1] / denom),
        jnp.mean(out[:, 2] / denom),
    ])


# speedup vs baseline: 17.1245x; 1.0439x over previous
import jax
import jax.numpy as jnp
from jax.experimental import pallas as pl
from jax.experimental.pallas import tpu as pltpu

_MAX_POS = 0.05
_MAX_ANG = 0.1
_ALPHA = 0.95
_POS2 = _MAX_POS * _MAX_POS
_NEG2 = (1.5 * _MAX_POS) ** 2
_ANG1 = _MAX_ANG
_ANG15 = 1.5 * _MAX_ANG
_CH = 16


def _make_body(N, C, M, RT):
    nch = RT // _CH

    def _body(cls_ref, reg_ref, anch_ref, ann_ref, out_ref):
        def chunk(i, accs):
            acc_cls, acc_xy, acc_ang, acc_np = accs
            sl = pl.ds(i * _CH, _CH)
            ax = anch_ref[0, sl, :]
            ay = anch_ref[1, sl, :]
            aa = anch_ref[2, sl, :]

            streams = []
            half = (M + 1) // 2
            for lo, hi in ((0, half), (half, M)):
                ms = jnp.full((_CH, 128), 1e30, jnp.float32)
                txs = jnp.zeros((_CH, 128), jnp.float32)
                tys = jnp.zeros((_CH, 128), jnp.float32)
                tas = jnp.zeros((_CH, 128), jnp.float32)
                tcs = jnp.zeros((_CH, 128), jnp.float32)
                for j in range(lo, hi):
                    txj = ann_ref[0, 0, j]
                    tyj = ann_ref[0, 1, j]
                    taj = ann_ref[0, 2, j]
                    tcj = ann_ref[0, 3, j]
                    dx = ax - txj
                    dy = ay - tyj
                    d2 = dx * dx + dy * dy
                    pred = d2 < ms
                    ms = jnp.where(pred, d2, ms)
                    txs = jnp.where(pred, txj, txs)
                    tys = jnp.where(pred, tyj, tys)
                    tas = jnp.where(pred, taj, tas)
                    tcs = jnp.where(pred, tcj, tcs)
                streams.append((ms, txs, tys, tas, tcs))
            (m0, tx0, ty0, ta0, tc0), (m1, tx1, ty1, ta1, tc1) = streams
            pred = m1 < m0
            m = jnp.where(pred, m1, m0)
            tx = jnp.where(pred, tx1, tx0)
            ty = jnp.where(pred, ty1, ty0)
            ta = jnp.where(pred, ta1, ta0)
            tc = jnp.where(pred, tc1, tc0)

            row = jax.lax.broadcasted_iota(jnp.int32, (_CH, 128), 0)
            lane = jax.lax.broadcasted_iota(jnp.int32, (_CH, 128), 1)
            gmask = (i * (_CH * 128) + row * 128 + lane) < N

            a_dist = jnp.abs(aa - ta)
            posm = (m < _POS2) & (a_dist < _ANG1) & gmask
            w = (((m >= _NEG2) | (a_dist >= _ANG15)) | posm) & gmask
            icls = tc.astype(jnp.int32)

            zero = jnp.zeros((_CH, 128), jnp.float32)
            s0 = zero
            corr = zero
            for c in range(C):
                p = jnp.clip(cls_ref[c, 0, sl, :], 1e-4, 1.0 - 1e-4)
                l0 = (p * p) * jnp.log1p(-p) * (-(1.0 - _ALPHA))
                omp = 1.0 - p
                l1 = (omp * omp) * jnp.log(p) * (-_ALPHA)
                s0 = s0 + l0
                corr = corr + jnp.where(posm & (icls == c), l1 - l0, zero)
            acc_cls = acc_cls + jnp.where(w, s0, zero) + corr

            sl1 = []
            for c, (a_c, t_c) in enumerate(((ax, tx), (ay, ty), (aa, ta))):
                d = reg_ref[c, 0, sl, :] - (t_c - a_c)
                ad = jnp.abs(d)
                sl1.append(jnp.where(ad <= 1.0 / 9.0, 4.5 * ad * ad,
                                     ad - 0.5 / 9.0))
            acc_xy = acc_xy + jnp.where(posm, sl1[0] + sl1[1], zero)
            acc_ang = acc_ang + jnp.where(posm, sl1[2], zero)
            acc_np = acc_np + jnp.where(posm, 1.0, 0.0)
            return (acc_cls, acc_xy, acc_ang, acc_np)

        z = jnp.zeros((_CH, 128), jnp.float32)
        acc_cls, acc_xy, acc_ang, acc_np = jax.lax.fori_loop(
            0, nch, chunk, (z, z, z, z))
        out_ref[0, 0, 0] = jnp.sum(acc_cls)
        out_ref[0, 0, 1] = jnp.sum(acc_xy)
        out_ref[0, 0, 2] = jnp.sum(acc_ang)
        out_ref[0, 0, 3] = jnp.sum(acc_np)

    return _body


@jax.jit
def kernel(classifications, regressions, anchors, annotations):
    B, N, C = classifications.shape
    M = annotations.shape[1]
    RT = -(-N // 128)
    if RT % _CH:
        RT += _CH - RT % _CH
    NP = RT * 128

    clsT = jnp.pad(classifications.transpose(2, 0, 1),
                   ((0, 0), (0, 0), (0, NP - N)),
                   constant_values=0.5).reshape(C, B, RT, 128)
    regT = jnp.pad(regressions.transpose(2, 0, 1),
                   ((0, 0), (0, 0), (0, NP - N))).reshape(3, B, RT, 128)
    anchT = jnp.pad(anchors[0].transpose(1, 0),
                    ((0, 0), (0, NP - N)),
                    constant_values=9.0).reshape(3, RT, 128)
    annT = annotations.transpose(0, 2, 1)
    annT = annT.at[:, 0, :].set(
        jnp.where(annotations[:, :, 3] != -1.0, annotations[:, :, 0], 1e6))

    out = pl.pallas_call(
        _make_body(N, C, M, RT),
        grid=(B,),
        in_specs=[
            pl.BlockSpec((C, 1, RT, 128), lambda b: (0, b, 0, 0)),
            pl.BlockSpec((3, 1, RT, 128), lambda b: (0, b, 0, 0)),
            pl.BlockSpec((3, RT, 128), lambda b: (0, 0, 0)),
            pl.BlockSpec((1, 4, M), lambda b: (b, 0, 0),
                         memory_space=pltpu.SMEM),
        ],
        out_specs=pl.BlockSpec(
            (1, 1, 4), lambda b: (b, 0, 0), memory_space=pltpu.SMEM),
        out_shape=jax.ShapeDtypeStruct((B, 1, 4), jnp.float32),
    )(clsT, regT, anchT, annT)

    out = out[:, 0, :]
    denom = jnp.maximum(out[:, 3], 1.0)
    return jnp.stack([
        jnp.mean(out[:, 0] / denom),
        jnp.mean(out[:, 1] / denom),
        jnp.mean(out[:, 2] / denom),
    ])
